# baseline (speedup 1.0000x reference)
import functools

import jax
import jax.numpy as jnp
from jax import lax
from jax.experimental import pallas as pl
from jax.experimental.pallas import tpu as pltpu
from jax.experimental.pallas import tpu_sc as plsc

N_RATE = 6
N_GENRE = 25
N_DIRECTOR = 2186
N_YEAR = 81
EMB = 32
D = 2 + N_GENRE + N_DIRECTOR
B = 16384
BT = 1024

C0 = 2048
NJ = D - C0
NJP = ((NJ + 15) // 16) * 16
NW = 32
RW = B // NW
RC = 8


def _tc_main(x_ref, w_big_ref, w_rate_ref, w_year_ref, out_ref):
    xf = x_ref[...].astype(jnp.float32)
    big = jax.lax.dot_general(
        xf, w_big_ref[...],
        (((1,), (0,)), ((), ())),
        preferred_element_type=jnp.float32,
    )
    genre_sig = jax.nn.sigmoid(big[:, 0:EMB])
    dir_logits = big[:, EMB:2 * EMB]

    rate_idx = x_ref[:, 0:1]
    year_idx = x_ref[:, 1:2]
    oh_rate = (rate_idx == jax.lax.broadcasted_iota(jnp.int32, (1, N_RATE), 1)
               ).astype(jnp.float32)
    oh_year = (year_idx == jax.lax.broadcasted_iota(jnp.int32, (1, N_YEAR), 1)
               ).astype(jnp.float32)
    rate_emb = jax.lax.dot_general(
        oh_rate, w_rate_ref[...], (((1,), (0,)), ((), ())),
        preferred_element_type=jnp.float32)
    year_emb = jax.lax.dot_general(
        oh_year, w_year_ref[...], (((1,), (0,)), ((), ())),
        preferred_element_type=jnp.float32)

    out_ref[...] = jnp.concatenate(
        [rate_emb, year_emb, genre_sig, dir_logits], axis=1)


def _merge(a_ref, b_ref, out_ref):
    a = a_ref[...]
    dir_sig = jax.nn.sigmoid(a[:, 96:128] + b_ref[...])
    out_ref[...] = jnp.concatenate([a[:, 0:96], dir_sig], axis=1)


def _sc_body(x_hbm, wt_hbm, out_hbm, xbuf, wbuf, obuf):
    wid = lax.axis_index("s") * 2 + lax.axis_index("c")
    pltpu.sync_copy(wt_hbm, wbuf)

    def chunk(c, carry):
        b0 = wid * RW + c * RC
        start = b0 * D
        pltpu.sync_copy(x_hbm.at[pl.ds(start, RC * D)],
                        xbuf.at[pl.ds(0, RC * D)])
        for r in range(RC):
            rbase = r * D + C0

            def jloop(jc, accs, rbase=rbase):
                a0, a1 = accs
                xv = xbuf[pl.ds(rbase + jc * 16, 16)].astype(jnp.float32)
                wb = jc * 16 * 2 * 16
                for l in range(16):
                    xs = xv[l]
                    w0 = wbuf[pl.ds(wb + l * 32, 16)]
                    w1 = wbuf[pl.ds(wb + l * 32 + 16, 16)]
                    a0 = a0 + xs * w0
                    a1 = a1 + xs * w1
                return a0, a1

            acc0, acc1 = lax.fori_loop(
                0, NJP // 16, jloop,
                (jnp.zeros((16,), jnp.float32), jnp.zeros((16,), jnp.float32)))
            obuf[pl.ds(r * 32, 16)] = acc0
            obuf[pl.ds(r * 32 + 16, 16)] = acc1
        pltpu.sync_copy(obuf, out_hbm.at[pl.ds(b0 * 32, RC * 32)])
        return carry

    lax.fori_loop(0, RW // RC, chunk, 0)


def kernel(x, W_rate, W_year, W_genre, W_director):
    W_big = jnp.zeros((C0, 2 * EMB), jnp.float32)
    W_big = W_big.at[2:2 + N_GENRE, 0:EMB].set(W_genre)
    W_big = W_big.at[2 + N_GENRE:C0, EMB:].set(W_director[:C0 - 2 - N_GENRE])

    W_tail = jnp.zeros((NJP, EMB), jnp.float32)
    W_tail = W_tail.at[:NJ].set(W_director[C0 - 2 - N_GENRE:])
    w_tail_flat = W_tail.reshape(-1)

    x_flat = x.reshape(-1)

    mesh = plsc.VectorSubcoreMesh(
        core_axis_name="c", subcore_axis_name="s",
        num_cores=2, num_subcores=16)
    sc_fn = pl.kernel(
        _sc_body,
        out_type=jax.ShapeDtypeStruct((B * EMB,), jnp.float32),
        mesh=mesh,
        scratch_types=[
            pltpu.VMEM((RC * D + 32,), jnp.int32),
            pltpu.VMEM((NJP * EMB,), jnp.float32),
            pltpu.VMEM((RC * EMB,), jnp.float32),
        ],
    )
    sc_partial = sc_fn(x_flat, w_tail_flat).reshape(B, EMB)

    tc_out = pl.pallas_call(
        _tc_main,
        grid=(B // BT,),
        in_specs=[
            pl.BlockSpec((BT, C0), lambda i: (i, 0)),
            pl.BlockSpec((C0, 2 * EMB), lambda i: (0, 0)),
            pl.BlockSpec((N_RATE, EMB), lambda i: (0, 0)),
            pl.BlockSpec((N_YEAR, EMB), lambda i: (0, 0)),
        ],
        out_specs=pl.BlockSpec((BT, 4 * EMB), lambda i: (i, 0)),
        out_shape=jax.ShapeDtypeStruct((B, 4 * EMB), jnp.float32),
    )(x, W_big, W_rate, W_year)

    return pl.pallas_call(
        _merge,
        grid=(B // BT,),
        in_specs=[
            pl.BlockSpec((BT, 4 * EMB), lambda i: (i, 0)),
            pl.BlockSpec((BT, EMB), lambda i: (i, 0)),
        ],
        out_specs=pl.BlockSpec((BT, 4 * EMB), lambda i: (i, 0)),
        out_shape=jax.ShapeDtypeStruct((B, 4 * EMB), jnp.float32),
        input_output_aliases={0: 0},
    )(tc_out, sc_partial)


# speedup vs baseline: 11.0725x; 11.0725x over previous
import jax
import jax.numpy as jnp
from jax.experimental import pallas as pl

N_RATE = 6
N_GENRE = 25
N_DIRECTOR = 2186
N_YEAR = 81
EMB = 32
D = 2 + N_GENRE + N_DIRECTOR
BT = 2048


def _tile_kernel(xt_ref, w_big_ref, w_rate_ref, w_year_ref, out_ref):
    xf = xt_ref[...].astype(jnp.float32)
    big = jax.lax.dot_general(
        xf, w_big_ref[...],
        (((0,), (1,)), ((), ())),
        preferred_element_type=jnp.float32,
    )
    gd = jax.nn.sigmoid(big)

    rate_idx = xt_ref[0:1, :]
    year_idx = xt_ref[1:2, :]
    oh_rate = (rate_idx == jax.lax.broadcasted_iota(jnp.int32, (N_RATE, 1), 0)
               ).astype(jnp.float32)
    oh_year = (year_idx == jax.lax.broadcasted_iota(jnp.int32, (N_YEAR, 1), 0)
               ).astype(jnp.float32)
    rate_emb = jax.lax.dot_general(
        oh_rate, w_rate_ref[...], (((0,), (0,)), ((), ())),
        preferred_element_type=jnp.float32)
    year_emb = jax.lax.dot_general(
        oh_year, w_year_ref[...], (((0,), (1,)), ((), ())),
        preferred_element_type=jnp.float32)

    out_ref[...] = jnp.concatenate([rate_emb, year_emb, gd], axis=1)


def kernel(x, W_rate, W_year, W_genre, W_director):
    B = x.shape[0]
    xt = x.T
    W_top = jnp.pad(W_genre.T, ((0, 0), (2, D - 2 - N_GENRE)))
    W_bot = jnp.pad(W_director.T, ((0, 0), (2 + N_GENRE, 0)))
    W_big = jnp.concatenate([W_top, W_bot], axis=0)

    return pl.pallas_call(
        _tile_kernel,
        grid=(B // BT,),
        in_specs=[
            pl.BlockSpec((D, BT), lambda i: (0, i)),
            pl.BlockSpec((2 * EMB, D), lambda i: (0, 0)),
            pl.BlockSpec((N_RATE, EMB), lambda i: (0, 0)),
            pl.BlockSpec((EMB, N_YEAR), lambda i: (0, 0)),
        ],
        out_specs=pl.BlockSpec((BT, 4 * EMB), lambda i: (i, 0)),
        out_shape=jax.ShapeDtypeStruct((B, 4 * EMB), jnp.float32),
    )(xt, W_big, W_rate, W_year.T)
